# baseline (speedup 1.0000x reference)
import functools

import jax
import jax.numpy as jnp
from jax import lax
from jax.experimental import pallas as pl
from jax.experimental.pallas import tpu as pltpu
from jax.experimental.pallas import tpu_sc as plsc

DIM = 128
L = 16
NC, NS = 2, 16
NW = NC * NS
CHUNK = 125
CPAD = 128


def _sc_kernel_body(R0, R1, R2, BPW, NCHUNK,
                    ef_hbm, w0_hbm, w1_hbm, w2_hbm,
                    out_hbm, tbl_hbm,
                    ef_v, w0_v, w1_v, w2_v, t_v,
                    combo0_v, combo1_v, rows0_v, rows1_v,
                    gsem0, gsem1, ssem0, ssem1):
    NT = R0 * R1 * R2
    NTP = (NT + 7) // 8 * 8
    wid = lax.axis_index("s") * NC + lax.axis_index("c")
    base = wid * BPW

    combos = (combo0_v, combo1_v)
    rows = (rows0_v, rows1_v)
    gsems = (gsem0, gsem1)
    ssems = (ssem0, ssem1)

    pltpu.sync_copy(w0_hbm, w0_v)
    pltpu.sync_copy(w1_hbm, w1_v)
    pltpu.sync_copy(w2_hbm, w2_v)
    pltpu.sync_copy(ef_hbm.at[pl.ds(base, BPW)], ef_v)

    def build_row(r, _):
        a = jnp.minimum(r // (R1 * R2), R0 - 1)
        rem = r % (R1 * R2)
        b = jnp.minimum(rem // R2, R1 - 1)
        c = rem % R2
        for k in range(DIM // L):
            sl = pl.ds(k * L, L)
            t_v[r, sl] = w0_v[a, sl] + w1_v[b, sl] + w2_v[c, sl]
        return _

    lax.fori_loop(0, NTP, build_row, 0)

    pltpu.sync_copy(t_v, tbl_hbm.at[pl.ds(wid * NTP, NTP)])

    lanes = lax.iota(jnp.int32, L)
    col0 = jnp.zeros((L,), jnp.int32)
    col1 = jnp.ones((L,), jnp.int32)
    col2 = jnp.full((L,), 2, jnp.int32)
    tbl_base = wid * NTP

    def compute_combo(t, cv):
        off = t * CHUNK
        for g in range(CPAD // L):
            r16 = jnp.minimum(off + g * L + lanes, BPW - 1)
            i0 = plsc.load_gather(ef_v, [r16, col0])
            i1 = plsc.load_gather(ef_v, [r16, col1])
            i2 = plsc.load_gather(ef_v, [r16, col2])
            cv[pl.ds(g * L, L)] = i0 * (R1 * R2) + i1 * R2 + i2 + tbl_base

    def gather_args(b):
        return tbl_hbm.at[combos[b]], rows[b], gsems[b]

    def scatter_args(t, b):
        return (rows[b].at[pl.ds(0, CHUNK)],
                out_hbm.at[pl.ds(base + t * CHUNK, CHUNK)], ssems[b])

    for b in range(2):
        compute_combo(b, combos[b])
        pltpu.async_copy(*gather_args(b))

    def outer(j, _):
        for b in range(2):
            i = 2 * j + b
            pltpu.make_async_copy(*gather_args(b)).wait()
            pltpu.async_copy(*scatter_args(i, b))
            compute_combo(i + 2, combos[b])
            pltpu.make_async_copy(*scatter_args(i, b)).wait()
            pltpu.async_copy(*gather_args(b))
        return _

    lax.fori_loop(0, NCHUNK // 2 - 1, outer, 0)

    for b in range(2):
        i = NCHUNK - 2 + b
        pltpu.make_async_copy(*gather_args(b)).wait()
        pltpu.async_copy(*scatter_args(i, b))
        pltpu.make_async_copy(*scatter_args(i, b)).wait()


def kernel(edge_feat, W0, W1, W2):
    E = edge_feat.shape[0]
    R0, R1, R2 = W0.shape[0], W1.shape[0], W2.shape[0]
    NTP = (R0 * R1 * R2 + 7) // 8 * 8
    assert E % (NW * CHUNK) == 0
    BPW = E // NW
    NCHUNK = BPW // CHUNK
    assert NCHUNK % 2 == 0

    mesh = plsc.VectorSubcoreMesh(core_axis_name="c", subcore_axis_name="s")
    f = pl.kernel(
        functools.partial(_sc_kernel_body, R0, R1, R2, BPW, NCHUNK),
        out_type=(
            jax.ShapeDtypeStruct((E, DIM), jnp.float32),
            jax.ShapeDtypeStruct((NW * NTP, DIM), jnp.float32),
        ),
        mesh=mesh,
        compiler_params=pltpu.CompilerParams(
            needs_layout_passes=False, use_tc_tiling_on_sc=False),
        scratch_types=[
            pltpu.VMEM((BPW, 3), jnp.int32),
            pltpu.VMEM((R0, DIM), jnp.float32),
            pltpu.VMEM((R1, DIM), jnp.float32),
            pltpu.VMEM((R2, DIM), jnp.float32),
            pltpu.VMEM((NTP, DIM), jnp.float32),
            pltpu.VMEM((CPAD,), jnp.int32),
            pltpu.VMEM((CPAD,), jnp.int32),
            pltpu.VMEM((CPAD, DIM), jnp.float32),
            pltpu.VMEM((CPAD, DIM), jnp.float32),
            pltpu.SemaphoreType.DMA,
            pltpu.SemaphoreType.DMA,
            pltpu.SemaphoreType.DMA,
            pltpu.SemaphoreType.DMA,
        ],
    )
    out, _ = f(edge_feat, W0, W1, W2)
    return out


# speedup vs baseline: 1.1544x; 1.1544x over previous
import functools

import jax
import jax.numpy as jnp
from jax import lax
from jax.experimental import pallas as pl
from jax.experimental.pallas import tpu as pltpu
from jax.experimental.pallas import tpu_sc as plsc

DIM = 128
L = 16
NC, NS = 2, 16
NW = NC * NS
CHUNK = 250
NG = (CHUNK + L - 1) // L


def _sc_kernel_body(R0, R1, R2, BPW, NCHUNK,
                    ef_hbm, w0_hbm, w1_hbm, w2_hbm, out_hbm,
                    ef_v, w0_v, w1_v, w2_v, t_v,
                    combo0_v, combo1_v, rows0_v, rows1_v,
                    ssem0, ssem1):
    NT = R0 * R1 * R2
    wid = lax.axis_index("s") * NC + lax.axis_index("c")
    base = wid * BPW

    combos = (combo0_v, combo1_v)
    rows = (rows0_v, rows1_v)
    ssems = (ssem0, ssem1)

    pltpu.sync_copy(w0_hbm, w0_v)
    pltpu.sync_copy(w1_hbm, w1_v)
    pltpu.sync_copy(w2_hbm, w2_v)
    pltpu.sync_copy(ef_hbm.at[pl.ds(base * 3, BPW * 3)], ef_v)

    def build_row(r, _):
        a = r // (R1 * R2)
        rem = r % (R1 * R2)
        b = rem // R2
        c = rem % R2
        for k in range(DIM // L):
            sl = pl.ds(k * L, L)
            t_v[pl.ds(r * DIM + k * L, L)] = (
                w0_v[a, sl] + w1_v[b, sl] + w2_v[c, sl])
        return _

    lax.fori_loop(0, NT, build_row, 0)

    lanes = lax.iota(jnp.int32, L)
    cols = [k * L + lanes for k in range(DIM // L)]

    def build_chunk(t, b):
        off = t * CHUNK
        cv = combos[b]
        rv = rows[b]
        for g in range(NG):
            r16 = jnp.minimum(off + g * L + lanes, BPW - 1) * 3
            i0 = plsc.load_gather(ef_v, [r16])
            i1 = plsc.load_gather(ef_v, [r16 + 1])
            i2 = plsc.load_gather(ef_v, [r16 + 2])
            cv[pl.ds(g * L, L)] = (i0 * (R1 * R2) + i1 * R2 + i2) * DIM
        def edge(e, _):
            for u in range(2):
                ee = 2 * e + u
                csp = plsc.load_gather(cv, [jnp.full((L,), ee, jnp.int32)])
                for k in range(DIM // L):
                    rv[ee, pl.ds(k * L, L)] = (
                        plsc.load_gather(t_v, [csp + cols[k]]))
            return _

        lax.fori_loop(0, CHUNK // 2, edge, 0)

    def scatter_args(t, b):
        return (rows[b],
                out_hbm.at[pl.ds(base + t * CHUNK, CHUNK)], ssems[b])

    for b in range(2):
        build_chunk(b, b)
        pltpu.async_copy(*scatter_args(b, b))

    def outer(j, _):
        for b in range(2):
            i = 2 * j + b
            pltpu.make_async_copy(*scatter_args(i - 2, b)).wait()
            build_chunk(i, b)
            pltpu.async_copy(*scatter_args(i, b))
        return _

    lax.fori_loop(1, NCHUNK // 2, outer, 0)

    for b in range(2):
        pltpu.make_async_copy(*scatter_args(NCHUNK - 2 + b, b)).wait()


def kernel(edge_feat, W0, W1, W2):
    E = edge_feat.shape[0]
    R0, R1, R2 = W0.shape[0], W1.shape[0], W2.shape[0]
    NT = R0 * R1 * R2
    assert E % (NW * CHUNK) == 0 and CHUNK % 2 == 0
    BPW = E // NW
    NCHUNK = BPW // CHUNK
    assert NCHUNK % 2 == 0

    mesh = plsc.VectorSubcoreMesh(core_axis_name="c", subcore_axis_name="s")
    f = pl.kernel(
        functools.partial(_sc_kernel_body, R0, R1, R2, BPW, NCHUNK),
        out_type=jax.ShapeDtypeStruct((E, DIM), jnp.float32),
        mesh=mesh,
        compiler_params=pltpu.CompilerParams(
            needs_layout_passes=False, use_tc_tiling_on_sc=False),
        scratch_types=[
            pltpu.VMEM((BPW * 3,), jnp.int32),
            pltpu.VMEM((R0, DIM), jnp.float32),
            pltpu.VMEM((R1, DIM), jnp.float32),
            pltpu.VMEM((R2, DIM), jnp.float32),
            pltpu.VMEM((NT * DIM,), jnp.float32),
            pltpu.VMEM((NG * L,), jnp.int32),
            pltpu.VMEM((NG * L,), jnp.int32),
            pltpu.VMEM((CHUNK, DIM), jnp.float32),
            pltpu.VMEM((CHUNK, DIM), jnp.float32),
            pltpu.SemaphoreType.DMA,
            pltpu.SemaphoreType.DMA,
        ],
    )
    return f(edge_feat.reshape(E * 3), W0, W1, W2)
